# baseline (speedup 1.0000x reference)
import functools

import jax
import jax.numpy as jnp
from jax import lax
from jax.experimental import pallas as pl
from jax.experimental.pallas import tpu as pltpu
from jax.experimental.pallas import tpu_sc as plsc

EMBED = 64
NUM_CORES = 2
NUM_SUBCORES = 16
NW = NUM_CORES * NUM_SUBCORES
CB = 256


def _emb_kernel(batch, hist, table_hbm, idx_hbm, out_hbm,
                idx_v, rows0, rows1, col0, col1, sl, sg0, sg1, ss0, ss1):
    b_blk = batch // NW
    halves = b_blk // CB
    n_chunks = hist * halves
    n_pairs = n_chunks // 2
    wid = lax.axis_index("s") * NUM_CORES + lax.axis_index("c")
    b0 = pl.multiple_of(wid * b_blk, 8)

    pltpu.async_copy(idx_hbm.at[:, pl.ds(b0, b_blk)], idx_v, sl).wait()

    def g_copy(c, rows, sem):
        h = c // halves
        off = pl.multiple_of((c % halves) * CB, 8)
        return pltpu.make_async_copy(
            table_hbm.at[idx_v.at[h, pl.ds(off, CB)]], rows, sem)

    def s_copy(c, col, sem):
        h = c // halves
        off = pl.multiple_of((c % halves) * CB, 8)
        return pltpu.make_async_copy(
            col, out_hbm.at[h, :, pl.ds(b0 + off, CB)], sem)

    lane = lax.iota(jnp.int32, 16)
    rowvecs = [lane + 16 * bb for bb in range(CB // 16)]

    def transpose(rows, col):
        def erow(e, _):
            ev = jnp.full((16,), e, dtype=jnp.int32)
            for bb in range(CB // 16):
                v = plsc.load_gather(rows, [rowvecs[bb], ev])
                col[e, pl.ds(16 * bb, 16)] = v
            return ()
        lax.fori_loop(0, EMBED, erow, (), unroll=False)

    g_copy(0, rows0, sg0).start()

    def pair(p, _):
        c0 = p * 2
        c1 = c0 + 1
        g_copy(c1, rows1, sg1).start()
        g_copy(c0, rows0, sg0).wait()

        @pl.when(p > 0)
        def _():
            s_copy(c0, col0, ss0).wait()
        transpose(rows0, col0)
        s_copy(c0, col0, ss0).start()

        @pl.when(p + 1 < n_pairs)
        def _():
            g_copy(c0 + 2, rows0, sg0).start()

        g_copy(c1, rows1, sg1).wait()

        @pl.when(p > 0)
        def _():
            s_copy(c1, col1, ss1).wait()
        transpose(rows1, col1)
        s_copy(c1, col1, ss1).start()
        return ()

    lax.fori_loop(0, n_pairs, pair, (), unroll=False)
    s_copy(n_chunks - 2, col0, ss0).wait()
    s_copy(n_chunks - 1, col1, ss1).wait()


def kernel(indices, table):
    batch, hist = indices.shape
    idx_t = indices.T.astype(jnp.int32)

    mesh = plsc.VectorSubcoreMesh(
        core_axis_name="c", subcore_axis_name="s",
        num_cores=NUM_CORES, num_subcores=NUM_SUBCORES,
    )
    k = pl.kernel(
        functools.partial(_emb_kernel, batch, hist),
        out_type=jax.ShapeDtypeStruct((hist, EMBED, batch), jnp.float32),
        mesh=mesh,
        scratch_types=[
            pltpu.VMEM((hist, batch // NW), jnp.int32),
            pltpu.VMEM((CB, EMBED), jnp.float32),
            pltpu.VMEM((CB, EMBED), jnp.float32),
            pltpu.VMEM((EMBED, CB), jnp.float32),
            pltpu.VMEM((EMBED, CB), jnp.float32),
            pltpu.SemaphoreType.DMA,
            pltpu.SemaphoreType.DMA,
            pltpu.SemaphoreType.DMA,
            pltpu.SemaphoreType.DMA,
            pltpu.SemaphoreType.DMA,
        ],
        compiler_params=pltpu.CompilerParams(
            use_tc_tiling_on_sc=False, needs_layout_passes=False),
    )
    out3 = k(table, idx_t)
    return out3.transpose(2, 0, 1)


# speedup vs baseline: 1.3905x; 1.3905x over previous
import functools

import jax
import jax.numpy as jnp
from jax import lax
from jax.experimental import pallas as pl
from jax.experimental.pallas import tpu as pltpu
from jax.experimental.pallas import tpu_sc as plsc

EMBED = 64
NUM_CORES = 2
NUM_SUBCORES = 16
NW = NUM_CORES * NUM_SUBCORES
CB = 256


def _emb_kernel(batch, hist, table_hbm, idx_hbm, out_hbm,
                idx_v, rows0, rows1, col0, col1, sl, sg0, sg1, ss0, ss1):
    b_blk = batch // NW
    halves = b_blk // CB
    n_chunks = hist * halves
    n_pairs = n_chunks // 2
    wid = lax.axis_index("s") * NUM_CORES + lax.axis_index("c")
    b0 = pl.multiple_of(wid * b_blk, 8)

    pltpu.async_copy(idx_hbm.at[:, pl.ds(b0, b_blk)], idx_v, sl).wait()

    def g_copy(c, rows, sem):
        h = c // halves
        off = pl.multiple_of((c % halves) * CB, 8)
        return pltpu.make_async_copy(
            table_hbm.at[idx_v.at[h, pl.ds(off, CB)]], rows, sem)

    def s_copy(c, col, sem):
        h = c // halves
        off = pl.multiple_of((c % halves) * CB, 8)
        return pltpu.make_async_copy(
            col, out_hbm.at[h, :, pl.ds(b0 + off, CB)], sem)

    lane = lax.iota(jnp.int32, 16)
    rowvecs = [lane + 16 * bb for bb in range(CB // 16)]

    def transpose(rows, col):
        @plsc.parallel_loop(0, EMBED, step=1, unroll=2)
        def _(e):
            ev = jnp.full((16,), e, dtype=jnp.int32)
            for bb in range(CB // 16):
                v = plsc.load_gather(rows, [rowvecs[bb], ev])
                col[e, pl.ds(16 * bb, 16)] = v

    g_copy(0, rows0, sg0).start()

    def pair(p, _):
        c0 = p * 2
        c1 = c0 + 1
        g_copy(c1, rows1, sg1).start()
        g_copy(c0, rows0, sg0).wait()

        @pl.when(p > 0)
        def _():
            s_copy(c0, col0, ss0).wait()
        transpose(rows0, col0)
        s_copy(c0, col0, ss0).start()

        @pl.when(p + 1 < n_pairs)
        def _():
            g_copy(c0 + 2, rows0, sg0).start()

        g_copy(c1, rows1, sg1).wait()

        @pl.when(p > 0)
        def _():
            s_copy(c1, col1, ss1).wait()
        transpose(rows1, col1)
        s_copy(c1, col1, ss1).start()
        return ()

    lax.fori_loop(0, n_pairs, pair, (), unroll=False)
    s_copy(n_chunks - 2, col0, ss0).wait()
    s_copy(n_chunks - 1, col1, ss1).wait()


def kernel(indices, table):
    batch, hist = indices.shape
    idx_t = indices.T.astype(jnp.int32)

    mesh = plsc.VectorSubcoreMesh(
        core_axis_name="c", subcore_axis_name="s",
        num_cores=NUM_CORES, num_subcores=NUM_SUBCORES,
    )
    k = pl.kernel(
        functools.partial(_emb_kernel, batch, hist),
        out_type=jax.ShapeDtypeStruct((hist, EMBED, batch), jnp.float32),
        mesh=mesh,
        scratch_types=[
            pltpu.VMEM((hist, batch // NW), jnp.int32),
            pltpu.VMEM((CB, EMBED), jnp.float32),
            pltpu.VMEM((CB, EMBED), jnp.float32),
            pltpu.VMEM((EMBED, CB), jnp.float32),
            pltpu.VMEM((EMBED, CB), jnp.float32),
            pltpu.SemaphoreType.DMA,
            pltpu.SemaphoreType.DMA,
            pltpu.SemaphoreType.DMA,
            pltpu.SemaphoreType.DMA,
            pltpu.SemaphoreType.DMA,
        ],
        compiler_params=pltpu.CompilerParams(
            use_tc_tiling_on_sc=False, needs_layout_passes=False),
    )
    out3 = k(table, idx_t)
    return out3.transpose(2, 0, 1)


# speedup vs baseline: 2.1005x; 1.5106x over previous
import functools

import jax
import jax.numpy as jnp
from jax import lax
from jax.experimental import pallas as pl
from jax.experimental.pallas import tpu as pltpu
from jax.experimental.pallas import tpu_sc as plsc

EMBED = 64
NUM_CORES = 2
NUM_SUBCORES = 16
NW = NUM_CORES * NUM_SUBCORES
CB = 256
CP = CB + 1


def _emb_kernel(batch, hist, table_hbm, idx_hbm, out_hbm,
                idx_v, rows0, rows1, col0, col1, sl, sg0, sg1, ss0, ss1):
    b_blk = batch // NW
    halves = b_blk // CB
    n_chunks = hist * halves
    n_pairs = n_chunks // 2
    wid = lax.axis_index("s") * NUM_CORES + lax.axis_index("c")
    b0 = pl.multiple_of(wid * b_blk, 8)

    pltpu.async_copy(idx_hbm.at[:, pl.ds(b0, b_blk)], idx_v, sl).wait()

    def g_copy(c, rows, sem):
        h = c // halves
        off = pl.multiple_of((c % halves) * CB, 8)
        return pltpu.make_async_copy(
            table_hbm.at[idx_v.at[h, pl.ds(off, CB)]], rows, sem)

    def s_copy(c, col, sem):
        h = c // halves
        off = pl.multiple_of((c % halves) * CB, 8)
        return pltpu.make_async_copy(
            col.at[:, pl.ds(0, CB)], out_hbm.at[h, :, pl.ds(b0 + off, CB)], sem)

    lane = lax.iota(jnp.int32, 16)
    evecs = [lane + 16 * k for k in range(EMBED // 16)]

    def transpose(rows, col):
        @plsc.parallel_loop(0, CB, step=1, unroll=2)
        def _(j):
            jv = jnp.full((16,), j, dtype=jnp.int32)
            for k in range(EMBED // 16):
                v = rows[j, pl.ds(16 * k, 16)]
                plsc.store_scatter(col, [evecs[k], jv], v)

    g_copy(0, rows0, sg0).start()

    def pair(p, _):
        c0 = p * 2
        c1 = c0 + 1
        g_copy(c1, rows1, sg1).start()
        g_copy(c0, rows0, sg0).wait()

        @pl.when(p > 0)
        def _():
            s_copy(c0, col0, ss0).wait()
        transpose(rows0, col0)
        s_copy(c0, col0, ss0).start()

        @pl.when(p + 1 < n_pairs)
        def _():
            g_copy(c0 + 2, rows0, sg0).start()

        g_copy(c1, rows1, sg1).wait()

        @pl.when(p > 0)
        def _():
            s_copy(c1, col1, ss1).wait()
        transpose(rows1, col1)
        s_copy(c1, col1, ss1).start()
        return ()

    lax.fori_loop(0, n_pairs, pair, (), unroll=False)
    s_copy(n_chunks - 2, col0, ss0).wait()
    s_copy(n_chunks - 1, col1, ss1).wait()


def kernel(indices, table):
    batch, hist = indices.shape
    idx_t = indices.T.astype(jnp.int32)

    mesh = plsc.VectorSubcoreMesh(
        core_axis_name="c", subcore_axis_name="s",
        num_cores=NUM_CORES, num_subcores=NUM_SUBCORES,
    )
    k = pl.kernel(
        functools.partial(_emb_kernel, batch, hist),
        out_type=jax.ShapeDtypeStruct((hist, EMBED, batch), jnp.float32),
        mesh=mesh,
        scratch_types=[
            pltpu.VMEM((hist, batch // NW), jnp.int32),
            pltpu.VMEM((CB, EMBED), jnp.float32),
            pltpu.VMEM((CB, EMBED), jnp.float32),
            pltpu.VMEM((EMBED, CP), jnp.float32),
            pltpu.VMEM((EMBED, CP), jnp.float32),
            pltpu.SemaphoreType.DMA,
            pltpu.SemaphoreType.DMA,
            pltpu.SemaphoreType.DMA,
            pltpu.SemaphoreType.DMA,
            pltpu.SemaphoreType.DMA,
        ],
        compiler_params=pltpu.CompilerParams(
            use_tc_tiling_on_sc=False, needs_layout_passes=False),
    )
    out3 = k(table, idx_t)
    return out3.transpose(2, 0, 1)


# speedup vs baseline: 2.6728x; 1.2724x over previous
import functools

import jax
import jax.numpy as jnp
from jax import lax
from jax.experimental import pallas as pl
from jax.experimental.pallas import tpu as pltpu
from jax.experimental.pallas import tpu_sc as plsc

EMBED = 64
NUM_CORES = 2
NUM_SUBCORES = 16
NW = NUM_CORES * NUM_SUBCORES
CB = 256
CP = CB + 1


def _emb_kernel(batch, hist, table_hbm, idx_hbm, out_hbm,
                idx_v, rows0, rows1, col0, col1, sl, sg0, sg1, ss0, ss1):
    b_blk = batch // NW
    halves = b_blk // CB
    n_chunks = hist * halves
    n_pairs = n_chunks // 2
    wid = lax.axis_index("s") * NUM_CORES + lax.axis_index("c")
    b0 = pl.multiple_of(wid * b_blk, 8)

    pltpu.async_copy(idx_hbm.at[:, pl.ds(b0, b_blk)], idx_v, sl).wait()

    def g_copy(c, rows, sem):
        h = c // halves
        off = pl.multiple_of((c % halves) * CB, 8)
        return pltpu.make_async_copy(
            table_hbm.at[idx_v.at[h, pl.ds(off, CB)]], rows, sem)

    def s_copies(c, col, sem):
        h = c // halves
        off = (c % halves) * CB
        bt_abs = (b0 + off) // 128
        return [
            pltpu.make_async_copy(
                col.at[btp, :, :, pl.ds(0, 128)],
                out_hbm.at[h, :, bt_abs + btp, :, :], sem)
            for btp in range(CB // 128)
        ]

    def s_start(c, col, sem):
        for cp in s_copies(c, col, sem):
            cp.start()

    def s_wait(c, col, sem):
        for cp in s_copies(c, col, sem):
            cp.wait()

    lane = lax.iota(jnp.int32, 16)
    ei_v = lax.rem(lane, 8)
    et_vs = [lane // 8 + 2 * k for k in range(EMBED // 16)]

    def transpose(rows, col):
        @plsc.parallel_loop(0, CB, step=1, unroll=2)
        def _(j):
            bt_v = jnp.full((16,), j // 128, dtype=jnp.int32)
            bi_v = jnp.full((16,), j % 128, dtype=jnp.int32)
            for k in range(EMBED // 16):
                v = rows[j, pl.ds(16 * k, 16)]
                plsc.store_scatter(col, [bt_v, et_vs[k], ei_v, bi_v], v)

    g_copy(0, rows0, sg0).start()

    def pair(p, _):
        c0 = p * 2
        c1 = c0 + 1
        g_copy(c1, rows1, sg1).start()
        g_copy(c0, rows0, sg0).wait()

        @pl.when(p > 0)
        def _():
            s_wait(c0, col0, ss0)
        transpose(rows0, col0)
        s_start(c0, col0, ss0)

        @pl.when(p + 1 < n_pairs)
        def _():
            g_copy(c0 + 2, rows0, sg0).start()

        g_copy(c1, rows1, sg1).wait()

        @pl.when(p > 0)
        def _():
            s_wait(c1, col1, ss1)
        transpose(rows1, col1)
        s_start(c1, col1, ss1)
        return ()

    lax.fori_loop(0, n_pairs, pair, (), unroll=False)
    s_wait(n_chunks - 2, col0, ss0)
    s_wait(n_chunks - 1, col1, ss1)


def kernel(indices, table):
    batch, hist = indices.shape
    idx_t = indices.T.astype(jnp.int32)

    mesh = plsc.VectorSubcoreMesh(
        core_axis_name="c", subcore_axis_name="s",
        num_cores=NUM_CORES, num_subcores=NUM_SUBCORES,
    )
    k = pl.kernel(
        functools.partial(_emb_kernel, batch, hist),
        out_type=jax.ShapeDtypeStruct(
            (hist, EMBED // 8, batch // 128, 8, 128), jnp.float32),
        mesh=mesh,
        scratch_types=[
            pltpu.VMEM((hist, batch // NW), jnp.int32),
            pltpu.VMEM((CB, EMBED), jnp.float32),
            pltpu.VMEM((CB, EMBED), jnp.float32),
            pltpu.VMEM((CB // 128, EMBED // 8, 8, 129), jnp.float32),
            pltpu.VMEM((CB // 128, EMBED // 8, 8, 129), jnp.float32),
            pltpu.SemaphoreType.DMA,
            pltpu.SemaphoreType.DMA,
            pltpu.SemaphoreType.DMA,
            pltpu.SemaphoreType.DMA,
            pltpu.SemaphoreType.DMA,
        ],
        compiler_params=pltpu.CompilerParams(
            use_tc_tiling_on_sc=False, needs_layout_passes=False),
    )
    out5 = k(table, idx_t)
    out3 = out5.transpose(0, 1, 3, 2, 4).reshape(hist, EMBED, batch)
    return out3.transpose(2, 0, 1)
